# baseline (speedup 1.0000x reference)
import functools

import jax
import jax.numpy as jnp
from jax import lax
from jax.experimental import pallas as pl

F_DIM = 256
BIN_SIZE = 128
K = 16
HI = lax.Precision.HIGHEST


def _bins_body(x_ref, cb_ref, p_ref):
    b = pl.program_id(0)
    N = x_ref.shape[1]
    NB = N // BIN_SIZE
    xb = x_ref[0]
    cb = cb_ref[:, : NB // 2]
    mul = jnp.dot(xb, cb)
    cmul = jnp.concatenate([mul, -mul], axis=1)
    mx = jnp.max(cmul, axis=1, keepdims=True)
    io_nb = lax.broadcasted_iota(jnp.int32, (N, NB), 1)
    binc = jnp.min(jnp.where(cmul == mx, io_nb, NB), axis=1, keepdims=True)
    onehot = (binc == io_nb).astype(jnp.float32)
    tot = jnp.sum(onehot, axis=0, keepdims=True)
    lt = (io_nb < binc).astype(jnp.float32)
    offset = jnp.sum(lt * tot, axis=1, keepdims=True)
    parts = []
    CH = 128
    for c in range(N // CH):
        rowa = lax.broadcasted_iota(jnp.int32, (CH, N), 0) + c * CH
        cola = lax.broadcasted_iota(jnp.int32, (CH, N), 1)
        tri = (cola < rowa).astype(jnp.float32)
        cnt = jnp.dot(tri, onehot)
        oh_c = onehot[c * CH:(c + 1) * CH]
        parts.append(jnp.sum(oh_c * cnt, axis=1, keepdims=True))
    rank = jnp.concatenate(parts, axis=0)
    p = offset + rank
    p_ref[0] = p.astype(jnp.int32) + N * b


def _sim_topk_body(x_ref, prow_ref, pcol_ref, bval_ref, bdst_ref):
    b = pl.program_id(0)
    i = pl.program_id(1)
    N = x_ref.shape[1]
    L = BIN_SIZE
    qbase = N * b + L * i
    prow = prow_ref[0]
    pcol = pcol_ref[0]
    rowq = lax.broadcasted_iota(jnp.int32, (L, N), 0) + qbase
    O = (jnp.broadcast_to(prow, (L, N)) == rowq).astype(jnp.float32)
    xb = jnp.dot(O, x_ref[0], precision=HI)
    qrow = lax.broadcasted_iota(jnp.int32, (N, L), 1) + qbase
    Ot = (jnp.broadcast_to(pcol, (N, L)) == qrow).astype(jnp.float32)
    jrow = lax.broadcasted_iota(jnp.int32, (1, N), 1).astype(jnp.float32)
    inv_row = jnp.dot(jrow, Ot, precision=HI)
    dm = lax.dot_general(xb, xb, (((1,), (1,)), ((), ())))
    work = jax.nn.sigmoid(dm)
    colio = lax.broadcasted_iota(jnp.int32, (L, L), 1)
    vals, dsts = [], []
    for _ in range(K):
        m = jnp.max(work, axis=1, keepdims=True)
        eqm = work == m
        selidx = jnp.min(jnp.where(eqm, colio, L), axis=1, keepdims=True)
        sel = colio == selidx
        gdst = jnp.sum(jnp.where(sel, inv_row, 0.0), axis=1, keepdims=True)
        vals.append(m)
        dsts.append(gdst)
        work = jnp.where(sel, -1.0, work)
    bval_ref[0, 0] = jnp.concatenate(vals, axis=1)
    bdst_ref[0, 0] = jnp.concatenate(dsts, axis=1).astype(jnp.int32)


def _expand_body(pcol_ref, bval_ref, bdst_ref, out_ref):
    b = pl.program_id(0)
    c = pl.program_id(1)
    N = bval_ref.shape[1]
    L = BIN_SIZE
    pchunk = pcol_ref[0] - N * b
    qio = lax.broadcasted_iota(jnp.int32, (L, N), 1)
    G = (jnp.broadcast_to(pchunk, (L, N)) == qio).astype(jnp.float32)
    tblv = jnp.dot(G, bval_ref[0], precision=HI)
    tbld = jnp.dot(G, bdst_ref[0].astype(jnp.float32), precision=HI)
    tbldi = tbld.astype(jnp.int32)
    colio = lax.broadcasted_iota(jnp.int32, (L, N), 1)
    acc = jnp.zeros((L, N), jnp.float32)
    for k in range(K):
        acc = acc + jnp.where(colio == tbldi[:, k:k + 1], tblv[:, k:k + 1], 0.0)
    out_ref[0] = acc


def kernel(x, codebook):
    B, N, F = x.shape
    NB = N // BIN_SIZE
    L = BIN_SIZE

    p_col = pl.pallas_call(
        _bins_body,
        grid=(B,),
        in_specs=[
            pl.BlockSpec((1, N, F), lambda b: (b, 0, 0)),
            pl.BlockSpec(codebook.shape, lambda b: (0, 0)),
        ],
        out_specs=pl.BlockSpec((1, N, 1), lambda b: (b, 0, 0)),
        out_shape=jax.ShapeDtypeStruct((B, N, 1), jnp.int32),
    )(x, codebook)

    p_row = jnp.swapaxes(p_col, 1, 2)

    bval, bdst = pl.pallas_call(
        _sim_topk_body,
        grid=(B, NB),
        in_specs=[
            pl.BlockSpec((1, N, F), lambda b, i: (b, 0, 0)),
            pl.BlockSpec((1, 1, N), lambda b, i: (b, 0, 0)),
            pl.BlockSpec((1, N, 1), lambda b, i: (b, 0, 0)),
        ],
        out_specs=[
            pl.BlockSpec((1, 1, L, K), lambda b, i: (b, i, 0, 0)),
            pl.BlockSpec((1, 1, L, K), lambda b, i: (b, i, 0, 0)),
        ],
        out_shape=[
            jax.ShapeDtypeStruct((B, NB, L, K), jnp.float32),
            jax.ShapeDtypeStruct((B, NB, L, K), jnp.int32),
        ],
    )(x, p_row, p_col)

    bval_f = bval.reshape(B, N, K)
    bdst_f = bdst.reshape(B, N, K)

    out = pl.pallas_call(
        _expand_body,
        grid=(B, N // L),
        in_specs=[
            pl.BlockSpec((1, L, 1), lambda b, c: (b, c, 0)),
            pl.BlockSpec((1, N, K), lambda b, c: (b, 0, 0)),
            pl.BlockSpec((1, N, K), lambda b, c: (b, 0, 0)),
        ],
        out_specs=pl.BlockSpec((1, L, N), lambda b, c: (b, c, 0)),
        out_shape=jax.ShapeDtypeStruct((B, N, N), jnp.float32),
    )(p_col, bval_f, bdst_f)
    return out


# speedup vs baseline: 1.6594x; 1.6594x over previous
import jax
import jax.numpy as jnp
from jax import lax
from jax.experimental import pallas as pl
from jax.experimental.pallas import tpu as pltpu
from jax.experimental.pallas import tpu_sc as plsc

F_DIM = 256
BIN_SIZE = 128
K = 16
NC = 2
NS = 16
NW = NC * NS


def _bins_body(x_ref, cb_ref, p_ref):
    b = pl.program_id(0)
    N = x_ref.shape[1]
    NB = N // BIN_SIZE
    xb = x_ref[0]
    cb = cb_ref[:, : NB // 2]
    mul = jnp.dot(xb, cb)
    cmul = jnp.concatenate([mul, -mul], axis=1)
    mx = jnp.max(cmul, axis=1, keepdims=True)
    io_nb = lax.broadcasted_iota(jnp.int32, (N, NB), 1)
    binc = jnp.min(jnp.where(cmul == mx, io_nb, NB), axis=1, keepdims=True)
    onehot = (binc == io_nb).astype(jnp.float32)
    tot = jnp.sum(onehot, axis=0, keepdims=True)
    lt = (io_nb < binc).astype(jnp.float32)
    offset = jnp.sum(lt * tot, axis=1, keepdims=True)
    parts = []
    CH = 128
    for c in range(N // CH):
        rowa = lax.broadcasted_iota(jnp.int32, (CH, N), 0) + c * CH
        cola = lax.broadcasted_iota(jnp.int32, (CH, N), 1)
        tri = (cola < rowa).astype(jnp.float32)
        cnt = jnp.dot(tri, onehot)
        oh_c = onehot[c * CH:(c + 1) * CH]
        parts.append(jnp.sum(oh_c * cnt, axis=1, keepdims=True))
    rank = jnp.concatenate(parts, axis=0)
    p = offset + rank
    p_ref[0] = p.astype(jnp.int32) + N * b


def _sc_gather_body(x_hbm, p_hbm, xb_hbm, inv_hbm,
                    q0, q1, j0, j1, xrows, sem):
    wid = lax.axis_index("s") * NC + lax.axis_index("c")
    rows_per = x_hbm.shape[0] // NW
    n_batch = 2048
    base = wid * rows_per
    base_mod = lax.rem(base, n_batch)
    lq0 = pltpu.async_copy(p_hbm.at[pl.ds(base, BIN_SIZE)], q0, sem)
    lq1 = pltpu.async_copy(p_hbm.at[pl.ds(base + BIN_SIZE, BIN_SIZE)], q1, sem)
    lx = pltpu.async_copy(x_hbm.at[pl.ds(base, 2 * BIN_SIZE)], xrows, sem)
    for i in range(8):
        j0[pl.ds(16 * i, 16)] = lax.iota(jnp.int32, 16) + (base_mod + 16 * i)
        j1[pl.ds(16 * i, 16)] = lax.iota(jnp.int32, 16) + (base_mod + 128 + 16 * i)
    lq0.wait()
    lq1.wait()
    lx.wait()
    sc = [
        pltpu.async_copy(xrows.at[pl.ds(0, BIN_SIZE)], xb_hbm.at[q0], sem),
        pltpu.async_copy(xrows.at[pl.ds(BIN_SIZE, BIN_SIZE)], xb_hbm.at[q1], sem),
        pltpu.async_copy(j0, inv_hbm.at[q0], sem),
        pltpu.async_copy(j1, inv_hbm.at[q1], sem),
    ]
    for c in sc:
        c.wait()


def _sim_topk_body(xb_ref, inv_ref, bval_ref, bdst_ref):
    L = BIN_SIZE
    MB = xb_ref.shape[1]
    R = MB * L
    dms = [
        lax.dot_general(xb_ref[0, m], xb_ref[0, m], (((1,), (1,)), ((), ())))
        for m in range(MB)
    ]
    work = jax.nn.sigmoid(jnp.concatenate(dms, axis=0))
    colio = lax.broadcasted_iota(jnp.int32, (R, L), 1)
    inv_rows = [jnp.broadcast_to(inv_ref[0, m], (L, L)) for m in range(MB)]
    key = colio * 2048 + jnp.concatenate(inv_rows, axis=0)
    big = jnp.int32(L * 2048)
    vals, dsts = [], []
    for _ in range(K):
        m = jnp.max(work, axis=1, keepdims=True)
        eqm = work == m
        comb = jnp.min(jnp.where(eqm, key, big), axis=1, keepdims=True)
        selidx = lax.shift_right_logical(comb, 11)
        dsts.append(comb - selidx * 2048)
        vals.append(m)
        work = jnp.where(colio == selidx, -1.0, work)
    padf = jnp.zeros((R, L - K), jnp.float32)
    padi = jnp.zeros((R, L - K), jnp.int32)
    allv = jnp.concatenate(vals + [padf], axis=1)
    alld = jnp.concatenate(dsts + [padi], axis=1)
    bval_ref[0] = allv.reshape(MB, L, L)
    bdst_ref[0] = alld.reshape(MB, L, L)


def _sc_expand_body(p_hbm, bval_hbm, bdst_hbm, out_hbm,
                    q0, q1, zbuf, valbuf, dstbuf, valbuf1, dstbuf1,
                    fidx, fval, sem, sem2, z0, z1, z2, z3):
    wid = lax.axis_index("s") * NC + lax.axis_index("c")
    n_batch = 2048
    rows_per = 256
    base = wid * rows_per
    zero16 = jnp.zeros((16,), jnp.float32)
    zwords = 16 * 1024
    def zbody(i, _):
        for u in range(4):
            zbuf[pl.ds(64 * i + 16 * u, 16)] = zero16
        return 0
    lax.fori_loop(0, zwords // 64, zbody, 0)
    zsems = (z0, z1, z2, z3)
    zcopies = [
        pltpu.async_copy(
            zbuf, out_hbm.at[pl.ds(base * n_batch + zwords * i, zwords)],
            zsems[i // 8])
        for i in range(32)
    ]
    pltpu.sync_copy(p_hbm.at[pl.ds(base, BIN_SIZE)], q0)
    pltpu.sync_copy(p_hbm.at[pl.ds(base + BIN_SIZE, BIN_SIZE)], q1)
    gv0 = pltpu.async_copy(bval_hbm.at[q0], valbuf, sem2)
    gd0 = pltpu.async_copy(bdst_hbm.at[q0], dstbuf, sem2)
    gv1 = pltpu.async_copy(bval_hbm.at[q1], valbuf1, sem2)
    gd1 = pltpu.async_copy(bdst_hbm.at[q1], dstbuf1, sem2)
    for half, (gv, gd, vb, db) in ((0, (gv0, gd0, valbuf, dstbuf)),
                                   (1, (gv1, gd1, valbuf1, dstbuf1))):
        gv.wait()
        gd.wait()
        for grp in range(16):
            gslot = 16 * half + grp
            for r in range(8):
                row = 8 * grp + r
                g = base + BIN_SIZE * half + row
                d = db[row, pl.ds(0, K)]
                fidx[gslot, pl.ds(K * r, K)] = d + g * n_batch
                fval[gslot, pl.ds(K * r, K)] = vb[row, pl.ds(0, K)]
    scatters = []
    for gg in range(4):
        for c in zcopies[8 * gg:8 * (gg + 1)]:
            c.wait()
        for gslot in range(8 * gg, 8 * (gg + 1)):
            scatters.append(pltpu.async_copy(
                fval.at[gslot], out_hbm.at[fidx.at[gslot]], sem2))
    for c in scatters:
        c.wait()


def kernel(x, codebook):
    B, N, F = x.shape
    NB = N // BIN_SIZE
    L = BIN_SIZE
    BN = B * N

    p_col = pl.pallas_call(
        _bins_body,
        grid=(B,),
        in_specs=[
            pl.BlockSpec((1, N, F), lambda b: (b, 0, 0)),
            pl.BlockSpec(codebook.shape, lambda b: (0, 0)),
        ],
        out_specs=pl.BlockSpec((1, N, 1), lambda b: (b, 0, 0)),
        out_shape=jax.ShapeDtypeStruct((B, N, 1), jnp.int32),
    )(x, codebook)

    p_flat = p_col.reshape(BN)
    mesh = plsc.VectorSubcoreMesh(core_axis_name="c", subcore_axis_name="s")

    sc_gather = pl.kernel(
        _sc_gather_body, mesh=mesh,
        out_type=[
            jax.ShapeDtypeStruct((BN, F), jnp.float32),
            jax.ShapeDtypeStruct((BN,), jnp.int32),
        ],
        scratch_types=[
            pltpu.VMEM((L,), jnp.int32),
            pltpu.VMEM((L,), jnp.int32),
            pltpu.VMEM((L,), jnp.int32),
            pltpu.VMEM((L,), jnp.int32),
            pltpu.VMEM((2 * L, F), jnp.float32),
            pltpu.SemaphoreType.DMA,
        ],
    )
    xb_flat, inv_flat = sc_gather(x.reshape(BN, F), p_flat)

    MB = 16
    bval, bdst = pl.pallas_call(
        _sim_topk_body,
        grid=(B, NB // MB),
        in_specs=[
            pl.BlockSpec((1, MB, L, F), lambda b, i: (b, i, 0, 0)),
            pl.BlockSpec((1, MB, 1, L), lambda b, i: (b, i, 0, 0)),
        ],
        out_specs=[
            pl.BlockSpec((1, MB, L, L), lambda b, i: (b, i, 0, 0)),
            pl.BlockSpec((1, MB, L, L), lambda b, i: (b, i, 0, 0)),
        ],
        out_shape=[
            jax.ShapeDtypeStruct((B, NB, L, L), jnp.float32),
            jax.ShapeDtypeStruct((B, NB, L, L), jnp.int32),
        ],
    )(xb_flat.reshape(B, NB, L, F), inv_flat.reshape(B, NB, 1, L))

    sc_expand = pl.kernel(
        _sc_expand_body, mesh=mesh,
        out_type=jax.ShapeDtypeStruct((BN * N,), jnp.float32),
        scratch_types=[
            pltpu.VMEM((L,), jnp.int32),
            pltpu.VMEM((L,), jnp.int32),
            pltpu.VMEM((16 * 1024,), jnp.float32),
            pltpu.VMEM((L, L), jnp.float32),
            pltpu.VMEM((L, L), jnp.int32),
            pltpu.VMEM((L, L), jnp.float32),
            pltpu.VMEM((L, L), jnp.int32),
            pltpu.VMEM((32, L), jnp.int32),
            pltpu.VMEM((32, L), jnp.float32),
            pltpu.SemaphoreType.DMA,
            pltpu.SemaphoreType.DMA,
            pltpu.SemaphoreType.DMA,
            pltpu.SemaphoreType.DMA,
            pltpu.SemaphoreType.DMA,
            pltpu.SemaphoreType.DMA,
        ],
    )
    out_flat = sc_expand(p_flat, bval.reshape(BN, L), bdst.reshape(BN, L))
    return out_flat.reshape(B, N, N)


# speedup vs baseline: 1.6747x; 1.0092x over previous
import jax
import jax.numpy as jnp
from jax import lax
from jax.experimental import pallas as pl
from jax.experimental.pallas import tpu as pltpu
from jax.experimental.pallas import tpu_sc as plsc

F_DIM = 256
BIN_SIZE = 128
K = 16
NC = 2
NS = 16
NW = NC * NS


def _bins_body(x_ref, cb_ref, p_ref):
    b = pl.program_id(0)
    N = x_ref.shape[1]
    NB = N // BIN_SIZE
    xb = x_ref[0]
    cb = cb_ref[:, : NB // 2]
    mul = jnp.dot(xb, cb)
    cmul = jnp.concatenate([mul, -mul], axis=1)
    mx = jnp.max(cmul, axis=1, keepdims=True)
    io_nb = lax.broadcasted_iota(jnp.int32, (N, NB), 1)
    binc = jnp.min(jnp.where(cmul == mx, io_nb, NB), axis=1, keepdims=True)
    onehot = (binc == io_nb).astype(jnp.float32)
    tot = jnp.sum(onehot, axis=0, keepdims=True)
    lt = (io_nb < binc).astype(jnp.float32)
    offset = jnp.sum(lt * tot, axis=1, keepdims=True)
    parts = []
    CH = 128
    for c in range(N // CH):
        rowa = lax.broadcasted_iota(jnp.int32, (CH, N), 0) + c * CH
        cola = lax.broadcasted_iota(jnp.int32, (CH, N), 1)
        tri = (cola < rowa).astype(jnp.float32)
        cnt = jnp.dot(tri, onehot)
        oh_c = onehot[c * CH:(c + 1) * CH]
        parts.append(jnp.sum(oh_c * cnt, axis=1, keepdims=True))
    rank = jnp.concatenate(parts, axis=0)
    p = offset + rank
    p_ref[0] = p.astype(jnp.int32) + N * b


def _sc_gather_body(x_hbm, p_hbm, xb_hbm, inv_hbm,
                    q0, q1, j0, j1, xrows, sem):
    wid = lax.axis_index("s") * NC + lax.axis_index("c")
    rows_per = x_hbm.shape[0] // NW
    n_batch = 2048
    base = wid * rows_per
    base_mod = lax.rem(base, n_batch)
    lq0 = pltpu.async_copy(p_hbm.at[pl.ds(base, BIN_SIZE)], q0, sem)
    lq1 = pltpu.async_copy(p_hbm.at[pl.ds(base + BIN_SIZE, BIN_SIZE)], q1, sem)
    lx = pltpu.async_copy(x_hbm.at[pl.ds(base, 2 * BIN_SIZE)], xrows, sem)
    for i in range(8):
        j0[pl.ds(16 * i, 16)] = lax.iota(jnp.int32, 16) + (base_mod + 16 * i)
        j1[pl.ds(16 * i, 16)] = lax.iota(jnp.int32, 16) + (base_mod + 128 + 16 * i)
    lq0.wait()
    lq1.wait()
    lx.wait()
    sc = [
        pltpu.async_copy(xrows.at[pl.ds(0, BIN_SIZE)], xb_hbm.at[q0], sem),
        pltpu.async_copy(xrows.at[pl.ds(BIN_SIZE, BIN_SIZE)], xb_hbm.at[q1], sem),
        pltpu.async_copy(j0, inv_hbm.at[q0], sem),
        pltpu.async_copy(j1, inv_hbm.at[q1], sem),
    ]
    for c in sc:
        c.wait()


def _sim_topk_body(xb_ref, inv_ref, bval_ref, bdst_ref):
    L = BIN_SIZE
    MB = xb_ref.shape[1]
    R = MB * L
    dms = [
        lax.dot_general(xb_ref[0, m], xb_ref[0, m], (((1,), (1,)), ((), ())))
        for m in range(MB)
    ]
    work = jax.nn.sigmoid(jnp.concatenate(dms, axis=0))
    colio = lax.broadcasted_iota(jnp.int32, (R, L), 1)
    inv_rows = [jnp.broadcast_to(inv_ref[0, m], (L, L)) for m in range(MB)]
    key = colio * 2048 + jnp.concatenate(inv_rows, axis=0)
    big = jnp.int32(L * 2048)
    vals, dsts = [], []
    for _ in range(K):
        m = jnp.max(work, axis=1, keepdims=True)
        eqm = work == m
        comb = jnp.min(jnp.where(eqm, key, big), axis=1, keepdims=True)
        selidx = lax.shift_right_logical(comb, 11)
        dsts.append(comb - selidx * 2048)
        vals.append(m)
        work = jnp.where(colio == selidx, -1.0, work)
    padf = jnp.zeros((R, L - K), jnp.float32)
    padi = jnp.zeros((R, L - K), jnp.int32)
    allv = jnp.concatenate(vals + [padf], axis=1)
    alld = jnp.concatenate(dsts + [padi], axis=1)
    bval_ref[0] = allv.reshape(MB, L, L)
    bdst_ref[0] = alld.reshape(MB, L, L)


def _sc_expand_body(p_hbm, bval_hbm, bdst_hbm, out_hbm,
                    q0, q1, zbuf, valbuf, dstbuf, valbuf1, dstbuf1,
                    fidx, fval, sem, sem2):
    wid = lax.axis_index("s") * NC + lax.axis_index("c")
    n_batch = 2048
    rows_per = 256
    base = wid * rows_per
    zero16 = jnp.zeros((16,), jnp.float32)
    zwords = 16 * 1024
    def zbody(i, _):
        for u in range(4):
            zbuf[pl.ds(64 * i + 16 * u, 16)] = zero16
        return 0
    lax.fori_loop(0, zwords // 64, zbody, 0)
    zcopies = [
        pltpu.async_copy(
            zbuf, out_hbm.at[pl.ds(base * n_batch + zwords * i, zwords)],
            sem)
        for i in range(32)
    ]
    pltpu.sync_copy(p_hbm.at[pl.ds(base, BIN_SIZE)], q0)
    pltpu.sync_copy(p_hbm.at[pl.ds(base + BIN_SIZE, BIN_SIZE)], q1)
    gv0 = pltpu.async_copy(bval_hbm.at[q0], valbuf, sem2)
    gd0 = pltpu.async_copy(bdst_hbm.at[q0], dstbuf, sem2)
    gv1 = pltpu.async_copy(bval_hbm.at[q1], valbuf1, sem2)
    gd1 = pltpu.async_copy(bdst_hbm.at[q1], dstbuf1, sem2)
    for half, (gv, gd, vb, db) in ((0, (gv0, gd0, valbuf, dstbuf)),
                                   (1, (gv1, gd1, valbuf1, dstbuf1))):
        gv.wait()
        gd.wait()
        for grp in range(16):
            gslot = 16 * half + grp
            for r in range(8):
                row = 8 * grp + r
                g = base + BIN_SIZE * half + row
                d = db[row, pl.ds(0, K)]
                fidx[gslot, pl.ds(K * r, K)] = d + g * n_batch
                fval[gslot, pl.ds(K * r, K)] = vb[row, pl.ds(0, K)]
    for c in zcopies:
        c.wait()
    scatters = [
        pltpu.async_copy(fval.at[gslot], out_hbm.at[fidx.at[gslot]], sem2)
        for gslot in range(32)
    ]
    for c in scatters:
        c.wait()


def kernel(x, codebook):
    B, N, F = x.shape
    NB = N // BIN_SIZE
    L = BIN_SIZE
    BN = B * N

    p_col = pl.pallas_call(
        _bins_body,
        grid=(B,),
        in_specs=[
            pl.BlockSpec((1, N, F), lambda b: (b, 0, 0)),
            pl.BlockSpec(codebook.shape, lambda b: (0, 0)),
        ],
        out_specs=pl.BlockSpec((1, N, 1), lambda b: (b, 0, 0)),
        out_shape=jax.ShapeDtypeStruct((B, N, 1), jnp.int32),
    )(x, codebook)

    p_flat = p_col.reshape(BN)
    mesh = plsc.VectorSubcoreMesh(core_axis_name="c", subcore_axis_name="s")

    sc_gather = pl.kernel(
        _sc_gather_body, mesh=mesh,
        out_type=[
            jax.ShapeDtypeStruct((BN, F), jnp.float32),
            jax.ShapeDtypeStruct((BN,), jnp.int32),
        ],
        scratch_types=[
            pltpu.VMEM((L,), jnp.int32),
            pltpu.VMEM((L,), jnp.int32),
            pltpu.VMEM((L,), jnp.int32),
            pltpu.VMEM((L,), jnp.int32),
            pltpu.VMEM((2 * L, F), jnp.float32),
            pltpu.SemaphoreType.DMA,
        ],
    )
    xb_flat, inv_flat = sc_gather(x.reshape(BN, F), p_flat)

    MB = 16
    bval, bdst = pl.pallas_call(
        _sim_topk_body,
        grid=(B, NB // MB),
        in_specs=[
            pl.BlockSpec((1, MB, L, F), lambda b, i: (b, i, 0, 0)),
            pl.BlockSpec((1, MB, 1, L), lambda b, i: (b, i, 0, 0)),
        ],
        out_specs=[
            pl.BlockSpec((1, MB, L, L), lambda b, i: (b, i, 0, 0)),
            pl.BlockSpec((1, MB, L, L), lambda b, i: (b, i, 0, 0)),
        ],
        out_shape=[
            jax.ShapeDtypeStruct((B, NB, L, L), jnp.float32),
            jax.ShapeDtypeStruct((B, NB, L, L), jnp.int32),
        ],
    )(xb_flat.reshape(B, NB, L, F), inv_flat.reshape(B, NB, 1, L))

    sc_expand = pl.kernel(
        _sc_expand_body, mesh=mesh,
        out_type=jax.ShapeDtypeStruct((BN * N,), jnp.float32),
        scratch_types=[
            pltpu.VMEM((L,), jnp.int32),
            pltpu.VMEM((L,), jnp.int32),
            pltpu.VMEM((16 * 1024,), jnp.float32),
            pltpu.VMEM((L, L), jnp.float32),
            pltpu.VMEM((L, L), jnp.int32),
            pltpu.VMEM((L, L), jnp.float32),
            pltpu.VMEM((L, L), jnp.int32),
            pltpu.VMEM((32, L), jnp.int32),
            pltpu.VMEM((32, L), jnp.float32),
            pltpu.SemaphoreType.DMA,
            pltpu.SemaphoreType.DMA,
        ],
    )
    out_flat = sc_expand(p_flat, bval.reshape(BN, L), bdst.reshape(BN, L))
    return out_flat.reshape(B, N, N)
